# Initial kernel scaffold; baseline (speedup 1.0000x reference)
import jax
import jax.numpy as jnp
from jax.experimental import pallas as pl


def kernel(image_feats, text_feats, image_preference, text_preference, W_img, b_img, W_txt, b_txt, edge_index):
    raise NotImplementedError("write your pallas kernel here")


# speedup vs baseline: 7.2786x; 7.2786x over previous
import functools

import jax
import jax.numpy as jnp
from jax import lax
from jax.experimental import pallas as pl
from jax.experimental.pallas import tpu as pltpu
from jax.experimental.pallas import tpu_sc as plsc

N_USERS = 5000
N_ITEMS = 5000
N = N_USERS + N_ITEMS
D = 128
E = 320000
N_LAYERS = 2
ALPHA = 0.5

NC = 2
NS = 16
CH = 128
NPAD = 10240
ROWS_PER_SUB = NPAD // NS
EPC = -(-E // (NS * CH)) * CH
E_PAD = EPC * NS
N_CHUNKS = EPC // CH

_MESH = plsc.VectorSubcoreMesh(core_axis_name="c", subcore_axis_name="s",
                               num_cores=NC, num_subcores=NS)


def _mm_norm_body(x_ref, w_ref, b_ref, o_ref):
    y = jnp.dot(x_ref[...], w_ref[...], preferred_element_type=jnp.float32)
    y = y + b_ref[...]
    o_ref[...] = y * lax.rsqrt(jnp.sum(y * y, axis=1, keepdims=True) + 1e-12)


def _mm_norm(feats, w, b, bm):
    m, k = feats.shape
    return pl.pallas_call(
        _mm_norm_body,
        grid=(m // bm,),
        in_specs=[
            pl.BlockSpec((bm, k), lambda i: (i, 0)),
            pl.BlockSpec((k, D), lambda i: (0, 0)),
            pl.BlockSpec((1, D), lambda i: (0, 0)),
        ],
        out_specs=pl.BlockSpec((bm, D), lambda i: (i, 0)),
        out_shape=jax.ShapeDtypeStruct((m, D), jnp.float32),
    )(feats, w, b.reshape(1, D))


def _prescale_body(x_ref, deg_ref, o_ref):
    f = lax.rsqrt(jnp.maximum(deg_ref[...], 1.0))
    o_ref[...] = x_ref[...] * f


def _prescale(x, deg_col):
    bm = 2048
    return pl.pallas_call(
        _prescale_body,
        grid=(x.shape[0] // bm,),
        in_specs=[pl.BlockSpec((bm, D), lambda i: (i, 0)),
                  pl.BlockSpec((bm, 1), lambda i: (i, 0))],
        out_specs=pl.BlockSpec((bm, D), lambda i: (i, 0)),
        out_shape=jax.ShapeDtypeStruct(x.shape, jnp.float32),
    )(x, deg_col)


def _post_body(u_ref, x_ref, dd_ref, ds_ref, xn_ref, tn_ref):
    g = lax.rsqrt(jnp.maximum(dd_ref[...], 1.0))
    xn = u_ref[...] * g + ALPHA * x_ref[...]
    xn_ref[...] = xn
    tn_ref[...] = xn * lax.rsqrt(jnp.maximum(ds_ref[...], 1.0))


def _postscale(u, x, dd_col, ds_col):
    bm = 2048
    return pl.pallas_call(
        _post_body,
        grid=(x.shape[0] // bm,),
        in_specs=[pl.BlockSpec((bm, D), lambda i: (i, 0)),
                  pl.BlockSpec((bm, D), lambda i: (i, 0)),
                  pl.BlockSpec((bm, 1), lambda i: (i, 0)),
                  pl.BlockSpec((bm, 1), lambda i: (i, 0))],
        out_specs=[pl.BlockSpec((bm, D), lambda i: (i, 0)),
                   pl.BlockSpec((bm, D), lambda i: (i, 0))],
        out_shape=[jax.ShapeDtypeStruct(x.shape, jnp.float32),
                   jax.ShapeDtypeStruct(x.shape, jnp.float32)],
    )(u, x, dd_col, ds_col)


def _deg_kernel_body(idx2_hbm, ones_hbm, zeros_hbm, deg_hbm, idx_v, ones_v, acc):
    c = lax.axis_index("c")
    s = lax.axis_index("s")
    pltpu.sync_copy(ones_hbm, ones_v)
    pltpu.sync_copy(zeros_hbm, acc.at[pl.ds(s * ROWS_PER_SUB, ROWS_PER_SUB)])
    plsc.subcore_barrier()

    @pl.loop(0, N_CHUNKS)
    def _(g):
        base = s * EPC + g * CH
        pltpu.sync_copy(idx2_hbm.at[c, pl.ds(base, CH)], idx_v)
        pltpu.sync_copy(ones_v, acc.at[idx_v], add=True)

    plsc.subcore_barrier()
    sl = pl.ds(s * ROWS_PER_SUB, ROWS_PER_SUB)
    pltpu.sync_copy(acc.at[sl], deg_hbm.at[c, sl])


def _degrees(idx2, ones_vec, zeros_vec):
    k = pl.kernel(
        _deg_kernel_body,
        out_type=jax.ShapeDtypeStruct((NC, NPAD), jnp.float32),
        mesh=_MESH,
        scratch_types=[
            pltpu.VMEM((CH,), jnp.int32),
            pltpu.VMEM((CH,), jnp.float32),
            pltpu.VMEM_SHARED((NPAD,), jnp.float32),
        ],
    )
    return k(idx2, ones_vec, zeros_vec)


def _spmm_kernel_body(t_hbm, src2_hbm, dst_hbm, zrows_hbm, u_hbm,
                      src_v, dst_v, rows_v, acc):
    c = lax.axis_index("c")
    s = lax.axis_index("s")
    pltpu.sync_copy(zrows_hbm, acc.at[pl.ds(s * ROWS_PER_SUB, ROWS_PER_SUB)])
    plsc.subcore_barrier()

    @pl.loop(0, N_CHUNKS)
    def _(g):
        base = s * EPC + g * CH
        pltpu.sync_copy(src2_hbm.at[c, pl.ds(base, CH)], src_v)
        pltpu.sync_copy(dst_hbm.at[pl.ds(base, CH)], dst_v)
        pltpu.sync_copy(t_hbm.at[src_v], rows_v)
        pltpu.sync_copy(rows_v, acc.at[dst_v], add=True)

    plsc.subcore_barrier()
    pltpu.sync_copy(acc.at[pl.ds(s * ROWS_PER_SUB, ROWS_PER_SUB)],
                    u_hbm.at[pl.ds(c * NPAD + s * ROWS_PER_SUB, ROWS_PER_SUB)])


def _spmm(t_both, src2, dst_p, zrows):
    k = pl.kernel(
        _spmm_kernel_body,
        out_type=jax.ShapeDtypeStruct((NC * NPAD, D), jnp.float32),
        mesh=_MESH,
        scratch_types=[
            pltpu.VMEM((CH,), jnp.int32),
            pltpu.VMEM((CH,), jnp.int32),
            pltpu.VMEM((CH, D), jnp.float32),
            pltpu.VMEM_SHARED((NPAD, D), jnp.float32),
        ],
    )
    return k(t_both, src2, dst_p, zrows)


def kernel(image_feats, text_feats, image_preference, text_preference,
           W_img, b_img, W_txt, b_txt, edge_index):
    src = edge_index[0]
    dst = edge_index[1]
    pad = E_PAD - E
    src_p = jnp.concatenate([src, jnp.full((pad,), N, jnp.int32)])
    dst_p = jnp.concatenate([dst, jnp.full((pad,), N, jnp.int32)])
    src2 = jnp.stack([src_p, src_p + NPAD])
    idx2 = jnp.stack([src_p, dst_p])

    ones_vec = jnp.ones((CH,), jnp.float32)
    zeros_vec = jnp.zeros((ROWS_PER_SUB,), jnp.float32)
    zrows = jnp.zeros((ROWS_PER_SUB, D), jnp.float32)

    deg2 = _degrees(idx2, ones_vec, zeros_vec)
    deg_src = jnp.concatenate([deg2[0], deg2[0]]).reshape(NC * NPAD, 1)
    deg_dst = jnp.concatenate([deg2[1], deg2[1]]).reshape(NC * NPAD, 1)

    img_emb = _mm_norm(image_feats, W_img, b_img, bm=200)
    txt_emb = _mm_norm(text_feats, W_txt, b_txt, bm=200)

    zpad = jnp.zeros((NPAD - N, D), jnp.float32)
    x = jnp.concatenate([image_preference, img_emb, zpad,
                         text_preference, txt_emb, zpad], axis=0)

    t = _prescale(x, deg_src)
    for layer in range(N_LAYERS):
        u = _spmm(t, src2, dst_p, zrows)
        x, t = _postscale(u, x, deg_dst, deg_src)

    user_preference = jnp.concatenate(
        [x[0:N_USERS], x[NPAD:NPAD + N_USERS]], axis=1)
    items = jnp.concatenate(
        [x[N_USERS:N], x[NPAD + N_USERS:NPAD + N]], axis=1)
    return (user_preference, items)


# speedup vs baseline: 8.1012x; 1.1130x over previous
import jax
import jax.numpy as jnp
from jax import lax
from jax.experimental import pallas as pl
from jax.experimental.pallas import tpu as pltpu
from jax.experimental.pallas import tpu_sc as plsc

N_USERS = 5000
N_ITEMS = 5000
N = N_USERS + N_ITEMS
D = 128
E = 320000
N_LAYERS = 2
ALPHA = 0.5

NC = 2
NS = 16
CH = 128
NB = 2
G = 8
NPAD = 10240
ROWS_PER_SUB = NPAD // NS
N_CHUNKS = -(-E // (NS * CH * NB)) * NB
EPC = N_CHUNKS * CH
E_PAD = EPC * NS
E_EXT = E_PAD + 4 * CH

_MESH = plsc.VectorSubcoreMesh(core_axis_name="c", subcore_axis_name="s",
                               num_cores=NC, num_subcores=NS)


def _mm_norm_body(x_ref, w_ref, b_ref, o_ref):
    y = jnp.dot(x_ref[...], w_ref[...], preferred_element_type=jnp.float32)
    y = y + b_ref[...]
    o_ref[...] = y * lax.rsqrt(jnp.sum(y * y, axis=1, keepdims=True) + 1e-12)


def _mm_norm(feats, w, b, bm):
    m, k = feats.shape
    return pl.pallas_call(
        _mm_norm_body,
        grid=(m // bm,),
        in_specs=[
            pl.BlockSpec((bm, k), lambda i: (i, 0)),
            pl.BlockSpec((k, D), lambda i: (0, 0)),
            pl.BlockSpec((1, D), lambda i: (0, 0)),
        ],
        out_specs=pl.BlockSpec((bm, D), lambda i: (i, 0)),
        out_shape=jax.ShapeDtypeStruct((m, D), jnp.float32),
    )(feats, w, b.reshape(1, D))


def _prescale_body(x_ref, deg_ref, o_ref):
    f = lax.rsqrt(jnp.maximum(deg_ref[...], 1.0))
    o_ref[...] = x_ref[...] * f


def _prescale(x, deg_col):
    bm = 2048
    return pl.pallas_call(
        _prescale_body,
        grid=(x.shape[0] // bm,),
        in_specs=[pl.BlockSpec((bm, D), lambda i: (i, 0)),
                  pl.BlockSpec((bm, 1), lambda i: (i, 0))],
        out_specs=pl.BlockSpec((bm, D), lambda i: (i, 0)),
        out_shape=jax.ShapeDtypeStruct(x.shape, jnp.float32),
    )(x, deg_col)


def _post_body(u_ref, x_ref, dd_ref, ds_ref, xn_ref, tn_ref):
    g = lax.rsqrt(jnp.maximum(dd_ref[...], 1.0))
    xn = u_ref[...] * g + ALPHA * x_ref[...]
    xn_ref[...] = xn
    tn_ref[...] = xn * lax.rsqrt(jnp.maximum(ds_ref[...], 1.0))


def _postscale(u, x, dd_col, ds_col):
    bm = 2048
    return pl.pallas_call(
        _post_body,
        grid=(x.shape[0] // bm,),
        in_specs=[pl.BlockSpec((bm, D), lambda i: (i, 0)),
                  pl.BlockSpec((bm, D), lambda i: (i, 0)),
                  pl.BlockSpec((bm, 1), lambda i: (i, 0)),
                  pl.BlockSpec((bm, 1), lambda i: (i, 0))],
        out_specs=[pl.BlockSpec((bm, D), lambda i: (i, 0)),
                   pl.BlockSpec((bm, D), lambda i: (i, 0))],
        out_shape=[jax.ShapeDtypeStruct(x.shape, jnp.float32),
                   jax.ShapeDtypeStruct(x.shape, jnp.float32)],
    )(u, x, dd_col, ds_col)


_DEG_SLOTS = 4


def _deg_kernel_body(idx2_hbm, ones_hbm, zeros_hbm, deg_hbm,
                     iv0, iv1, iv2, iv3, ones_v, acc, isems, ssems):
    ivs = (iv0, iv1, iv2, iv3)
    c = lax.axis_index("c")
    s = lax.axis_index("s")
    ebase = pl.multiple_of(s * EPC, CH)
    pltpu.sync_copy(ones_hbm, ones_v)
    zb = pl.multiple_of(s * ROWS_PER_SUB, 128)
    pltpu.sync_copy(zeros_hbm, acc.at[pl.ds(zb, ROWS_PER_SUB)])
    plsc.subcore_barrier()

    @pl.loop(0, N_CHUNKS)
    def _(g):
        gb = pl.multiple_of(ebase + g * CH, CH)
        pltpu.sync_copy(idx2_hbm.at[c, pl.ds(gb, CH)], iv0)
        pltpu.sync_copy(ones_v, acc.at[iv0], add=True)

    plsc.subcore_barrier()
    sl = pl.ds(zb, ROWS_PER_SUB)
    pltpu.sync_copy(acc.at[sl], deg_hbm.at[c, sl])


def _degrees(idx2, ones_vec, zeros_vec):
    k = pl.kernel(
        _deg_kernel_body,
        out_type=jax.ShapeDtypeStruct((NC, NPAD), jnp.float32),
        mesh=_MESH,
        scratch_types=[
            pltpu.VMEM((CH,), jnp.int32),
            pltpu.VMEM((CH,), jnp.int32),
            pltpu.VMEM((CH,), jnp.int32),
            pltpu.VMEM((CH,), jnp.int32),
            pltpu.VMEM((CH,), jnp.float32),
            pltpu.VMEM_SHARED((NPAD,), jnp.float32),
            pltpu.SemaphoreType.DMA((_DEG_SLOTS,)),
            pltpu.SemaphoreType.DMA((_DEG_SLOTS,)),
        ],
    )
    return k(idx2, ones_vec, zeros_vec)


def _spmm_kernel_body(t_hbm, src2_hbm, dst_hbm, zrows_hbm, u_hbm,
                      sv0, sv1, dv0, dv1, rows, acc, isems, gsems):
    svs = (sv0, sv1)
    dvs = (dv0, dv1)
    c = lax.axis_index("c")
    s = lax.axis_index("s")
    ebase = pl.multiple_of(s * EPC, CH)
    abase = pl.multiple_of(s * ROWS_PER_SUB, 128)
    pltpu.sync_copy(zrows_hbm, rows.at[0])
    for r in range(ROWS_PER_SUB // CH):
        pltpu.sync_copy(rows.at[0], acc.at[pl.ds(abase + r * CH, CH)])
    for b in range(NB):
        pltpu.async_copy(src2_hbm.at[c, pl.ds(ebase + b * CH, CH)],
                         svs[b], isems.at[b, 0])
        pltpu.async_copy(dst_hbm.at[pl.ds(ebase + b * CH, CH)],
                         dvs[b], isems.at[b, 1])
    plsc.subcore_barrier()

    @pl.loop(0, N_CHUNKS, step=NB)
    def _(g):
        for b in range(NB):
            pltpu.make_async_copy(src2_hbm.at[c, pl.ds(0, CH)], svs[b],
                                  isems.at[b, 0]).wait()
            pltpu.async_copy(t_hbm.at[svs[b]], rows.at[b], gsems.at[b])
        for b in range(NB):
            pltpu.make_async_copy(t_hbm.at[svs[b]], rows.at[b],
                                  gsems.at[b]).wait()
            pltpu.make_async_copy(dst_hbm.at[pl.ds(0, CH)], dvs[b],
                                  isems.at[b, 1]).wait()
            pltpu.sync_copy(rows.at[b], acc.at[dvs[b]], add=True)
            nxt = pl.multiple_of(ebase + (g + NB + b) * CH, CH)
            pltpu.async_copy(src2_hbm.at[c, pl.ds(nxt, CH)], svs[b],
                             isems.at[b, 0])
            pltpu.async_copy(dst_hbm.at[pl.ds(nxt, CH)], dvs[b],
                             isems.at[b, 1])

    for b in range(NB):
        pltpu.make_async_copy(src2_hbm.at[c, pl.ds(0, CH)], svs[b],
                              isems.at[b, 0]).wait()
        pltpu.make_async_copy(dst_hbm.at[pl.ds(0, CH)], dvs[b],
                              isems.at[b, 1]).wait()
    plsc.subcore_barrier()
    ob = pl.multiple_of(c * NPAD + s * ROWS_PER_SUB, 128)
    pltpu.sync_copy(acc.at[pl.ds(abase, ROWS_PER_SUB)],
                    u_hbm.at[pl.ds(ob, ROWS_PER_SUB)])


def _spmm(t_both, src2, dst_p, zrows):
    k = pl.kernel(
        _spmm_kernel_body,
        out_type=jax.ShapeDtypeStruct((NC * NPAD, D), jnp.float32),
        mesh=_MESH,
        scratch_types=[
            pltpu.VMEM((CH,), jnp.int32),
            pltpu.VMEM((CH,), jnp.int32),
            pltpu.VMEM((CH,), jnp.int32),
            pltpu.VMEM((CH,), jnp.int32),
            pltpu.VMEM((NB, CH, D), jnp.float32),
            pltpu.VMEM_SHARED((NPAD, D), jnp.float32),
            pltpu.SemaphoreType.DMA((NB, 2)),
            pltpu.SemaphoreType.DMA((NB,)),
        ],
    )
    return k(t_both, src2, dst_p, zrows)


def kernel(image_feats, text_feats, image_preference, text_preference,
           W_img, b_img, W_txt, b_txt, edge_index):
    src = edge_index[0]
    dst = edge_index[1]
    pad = E_EXT - E
    src_p = jnp.concatenate([src, jnp.full((pad,), N, jnp.int32)])
    dst_p = jnp.concatenate([dst, jnp.full((pad,), N, jnp.int32)])
    src2 = jnp.stack([src_p, src_p + NPAD])
    idx2 = jnp.stack([src_p, dst_p])

    ones_vec = jnp.ones((CH,), jnp.float32)
    zeros_vec = jnp.zeros((ROWS_PER_SUB,), jnp.float32)
    zrows = jnp.zeros((CH, D), jnp.float32)

    deg2 = _degrees(idx2, ones_vec, zeros_vec)
    deg_src = jnp.concatenate([deg2[0], deg2[0]]).reshape(NC * NPAD, 1)
    deg_dst = jnp.concatenate([deg2[1], deg2[1]]).reshape(NC * NPAD, 1)

    img_emb = _mm_norm(image_feats, W_img, b_img, bm=200)
    txt_emb = _mm_norm(text_feats, W_txt, b_txt, bm=200)

    zpad = jnp.zeros((NPAD - N, D), jnp.float32)
    x = jnp.concatenate([image_preference, img_emb, zpad,
                         text_preference, txt_emb, zpad], axis=0)

    t = _prescale(x, deg_src)
    for _layer in range(N_LAYERS):
        u = _spmm(t, src2, dst_p, zrows)
        x, t = _postscale(u, x, deg_dst, deg_src)

    user_preference = jnp.concatenate(
        [x[0:N_USERS], x[NPAD:NPAD + N_USERS]], axis=1)
    items = jnp.concatenate(
        [x[N_USERS:N], x[NPAD + N_USERS:NPAD + N]], axis=1)
    return (user_preference, items)


# speedup vs baseline: 8.2326x; 1.0162x over previous
import jax
import jax.numpy as jnp
from jax import lax
from jax.experimental import pallas as pl
from jax.experimental.pallas import tpu as pltpu
from jax.experimental.pallas import tpu_sc as plsc

N_USERS = 5000
N_ITEMS = 5000
N = N_USERS + N_ITEMS
D = 128
E = 320000
N_LAYERS = 2
ALPHA = 0.5

NC = 2
NS = 16
CH = 128
NB = 2
G = 8
NPAD = 10240
ROWS_PER_SUB = NPAD // NS
N_CHUNKS = -(-E // (NS * CH * NB)) * NB
EPC = N_CHUNKS * CH
E_PAD = EPC * NS
E_EXT = E_PAD + 8 * CH
NI = 2

_MESH = plsc.VectorSubcoreMesh(core_axis_name="c", subcore_axis_name="s",
                               num_cores=NC, num_subcores=NS)


def _mm_norm_body(x_ref, w_ref, b_ref, o_ref):
    y = jnp.dot(x_ref[...], w_ref[...], preferred_element_type=jnp.float32)
    y = y + b_ref[...]
    o_ref[...] = y * lax.rsqrt(jnp.sum(y * y, axis=1, keepdims=True) + 1e-12)


def _mm_norm(feats, w, b, bm):
    m, k = feats.shape
    return pl.pallas_call(
        _mm_norm_body,
        grid=(m // bm,),
        in_specs=[
            pl.BlockSpec((bm, k), lambda i: (i, 0)),
            pl.BlockSpec((k, D), lambda i: (0, 0)),
            pl.BlockSpec((1, D), lambda i: (0, 0)),
        ],
        out_specs=pl.BlockSpec((bm, D), lambda i: (i, 0)),
        out_shape=jax.ShapeDtypeStruct((m, D), jnp.float32),
    )(feats, w, b.reshape(1, D))


def _prescale_body(x_ref, deg_ref, o_ref):
    f = lax.rsqrt(jnp.maximum(deg_ref[...], 1.0))
    o_ref[...] = x_ref[...] * f


def _prescale(x, deg_col):
    bm = 2048
    return pl.pallas_call(
        _prescale_body,
        grid=(x.shape[0] // bm,),
        in_specs=[pl.BlockSpec((bm, D), lambda i: (i, 0)),
                  pl.BlockSpec((bm, 1), lambda i: (i, 0))],
        out_specs=pl.BlockSpec((bm, D), lambda i: (i, 0)),
        out_shape=jax.ShapeDtypeStruct(x.shape, jnp.float32),
    )(x, deg_col)


def _post_body(u_ref, x_ref, dd_ref, ds_ref, xn_ref, tn_ref):
    g = lax.rsqrt(jnp.maximum(dd_ref[...], 1.0))
    xn = u_ref[...] * g + ALPHA * x_ref[...]
    xn_ref[...] = xn
    tn_ref[...] = xn * lax.rsqrt(jnp.maximum(ds_ref[...], 1.0))


def _postscale(u, x, dd_col, ds_col):
    bm = 2048
    return pl.pallas_call(
        _post_body,
        grid=(x.shape[0] // bm,),
        in_specs=[pl.BlockSpec((bm, D), lambda i: (i, 0)),
                  pl.BlockSpec((bm, D), lambda i: (i, 0)),
                  pl.BlockSpec((bm, 1), lambda i: (i, 0)),
                  pl.BlockSpec((bm, 1), lambda i: (i, 0))],
        out_specs=[pl.BlockSpec((bm, D), lambda i: (i, 0)),
                   pl.BlockSpec((bm, D), lambda i: (i, 0))],
        out_shape=[jax.ShapeDtypeStruct(x.shape, jnp.float32),
                   jax.ShapeDtypeStruct(x.shape, jnp.float32)],
    )(u, x, dd_col, ds_col)


_DEG_SLOTS = 4


def _deg_kernel_body(idx2_hbm, ones_hbm, zeros_hbm, deg_hbm,
                     iv0, iv1, iv2, iv3, ones_v, acc, isems, ssems):
    ivs = (iv0, iv1, iv2, iv3)
    c = lax.axis_index("c")
    s = lax.axis_index("s")
    ebase = pl.multiple_of(s * EPC, CH)
    pltpu.sync_copy(ones_hbm, ones_v)
    zb = pl.multiple_of(s * ROWS_PER_SUB, 128)
    pltpu.sync_copy(zeros_hbm, acc.at[pl.ds(zb, ROWS_PER_SUB)])
    plsc.subcore_barrier()

    @pl.loop(0, N_CHUNKS)
    def _(g):
        gb = pl.multiple_of(ebase + g * CH, CH)
        pltpu.sync_copy(idx2_hbm.at[c, pl.ds(gb, CH)], iv0)
        pltpu.sync_copy(ones_v, acc.at[iv0], add=True)

    plsc.subcore_barrier()
    sl = pl.ds(zb, ROWS_PER_SUB)
    pltpu.sync_copy(acc.at[sl], deg_hbm.at[c, sl])


def _degrees(idx2, ones_vec, zeros_vec):
    k = pl.kernel(
        _deg_kernel_body,
        out_type=jax.ShapeDtypeStruct((NC, NPAD), jnp.float32),
        mesh=_MESH,
        scratch_types=[
            pltpu.VMEM((CH,), jnp.int32),
            pltpu.VMEM((CH,), jnp.int32),
            pltpu.VMEM((CH,), jnp.int32),
            pltpu.VMEM((CH,), jnp.int32),
            pltpu.VMEM((CH,), jnp.float32),
            pltpu.VMEM_SHARED((NPAD,), jnp.float32),
            pltpu.SemaphoreType.DMA((_DEG_SLOTS,)),
            pltpu.SemaphoreType.DMA((_DEG_SLOTS,)),
        ],
    )
    return k(idx2, ones_vec, zeros_vec)


def _spmm_kernel_body(t_hbm, src2_hbm, dst_hbm, zrows_hbm, u_hbm, *scr):
    svs = scr[0:NI]
    dvs = scr[NI:2 * NI]
    rows = scr[2 * NI]
    acc = scr[2 * NI + 1]
    isems = scr[2 * NI + 2:2 * NI + 2 + NI]
    dsems = scr[2 * NI + 2 + NI:2 * NI + 2 + 2 * NI]
    gsems = scr[2 * NI + 2 + 2 * NI:]
    c = lax.axis_index("c")
    s = lax.axis_index("s")
    ebase = pl.multiple_of(s * EPC, CH)
    abase = pl.multiple_of(s * ROWS_PER_SUB, 128)

    def refill(slot, q):
        nxt = pl.multiple_of(ebase + q * CH, CH)
        pltpu.async_copy(src2_hbm.at[c, pl.ds(nxt, CH)], svs[slot],
                         isems[slot])
        pltpu.async_copy(dst_hbm.at[pl.ds(nxt, CH)], dvs[slot],
                         dsems[slot])

    pltpu.sync_copy(zrows_hbm, rows.at[0])
    for r in range(ROWS_PER_SUB // CH):
        pltpu.sync_copy(rows.at[0], acc.at[pl.ds(abase + r * CH, CH)])
    for j in range(NI):
        refill(j, j)
    plsc.subcore_barrier()

    @pl.loop(0, N_CHUNKS, step=2)
    def _(g):
        for b in range(2):
            pltpu.make_async_copy(src2_hbm.at[c, pl.ds(0, CH)],
                                  svs[b], isems[b]).wait()
            pltpu.async_copy(t_hbm.at[svs[b]], rows.at[b], gsems[b])
        for b in range(2):
            pltpu.make_async_copy(t_hbm.at[svs[b]], rows.at[b],
                                  gsems[b]).wait()
            pltpu.make_async_copy(dst_hbm.at[pl.ds(0, CH)], dvs[b],
                                  dsems[b]).wait()
            pltpu.sync_copy(rows.at[b], acc.at[dvs[b]], add=True)
            refill(b, g + 2 + b)

    for j in range(NI):
        pltpu.make_async_copy(src2_hbm.at[c, pl.ds(0, CH)], svs[j],
                              isems[j]).wait()
        pltpu.make_async_copy(dst_hbm.at[pl.ds(0, CH)], dvs[j],
                              dsems[j]).wait()
    plsc.subcore_barrier()
    ob = pl.multiple_of(c * NPAD + s * ROWS_PER_SUB, 128)
    pltpu.sync_copy(acc.at[pl.ds(abase, ROWS_PER_SUB)],
                    u_hbm.at[pl.ds(ob, ROWS_PER_SUB)])


def _spmm(t_both, src2, dst_p, zrows):
    k = pl.kernel(
        _spmm_kernel_body,
        out_type=jax.ShapeDtypeStruct((NC * NPAD, D), jnp.float32),
        mesh=_MESH,
        scratch_types=(
            [pltpu.VMEM((CH,), jnp.int32) for _ in range(2 * NI)] + [
                pltpu.VMEM((2, CH, D), jnp.float32),
                pltpu.VMEM_SHARED((NPAD, D), jnp.float32),
            ] + [pltpu.SemaphoreType.DMA] * (2 * NI + 2)
        ),
    )
    return k(t_both, src2, dst_p, zrows)


def kernel(image_feats, text_feats, image_preference, text_preference,
           W_img, b_img, W_txt, b_txt, edge_index):
    src = edge_index[0]
    dst = edge_index[1]
    pad = E_EXT - E
    src_p = jnp.concatenate([src, jnp.full((pad,), N, jnp.int32)])
    dst_p = jnp.concatenate([dst, jnp.full((pad,), N, jnp.int32)])
    src2 = jnp.stack([src_p, src_p + NPAD])
    idx2 = jnp.stack([src_p, dst_p])

    ones_vec = jnp.ones((CH,), jnp.float32)
    zeros_vec = jnp.zeros((ROWS_PER_SUB,), jnp.float32)
    zrows = jnp.zeros((CH, D), jnp.float32)

    deg2 = _degrees(idx2, ones_vec, zeros_vec)
    deg_src = jnp.concatenate([deg2[0], deg2[0]]).reshape(NC * NPAD, 1)
    deg_dst = jnp.concatenate([deg2[1], deg2[1]]).reshape(NC * NPAD, 1)

    img_emb = _mm_norm(image_feats, W_img, b_img, bm=200)
    txt_emb = _mm_norm(text_feats, W_txt, b_txt, bm=200)

    zpad = jnp.zeros((NPAD - N, D), jnp.float32)
    x = jnp.concatenate([image_preference, img_emb, zpad,
                         text_preference, txt_emb, zpad], axis=0)

    t = _prescale(x, deg_src)
    for _layer in range(N_LAYERS):
        u = _spmm(t, src2, dst_p, zrows)
        x, t = _postscale(u, x, deg_dst, deg_src)

    user_preference = jnp.concatenate(
        [x[0:N_USERS], x[NPAD:NPAD + N_USERS]], axis=1)
    items = jnp.concatenate(
        [x[N_USERS:N], x[NPAD + N_USERS:NPAD + N]], axis=1)
    return (user_preference, items)


# speedup vs baseline: 8.5152x; 1.0343x over previous
import jax
import jax.numpy as jnp
from jax import lax
from jax.experimental import pallas as pl
from jax.experimental.pallas import tpu as pltpu
from jax.experimental.pallas import tpu_sc as plsc

N_USERS = 5000
N_ITEMS = 5000
N = N_USERS + N_ITEMS
D = 128
E = 320000
N_LAYERS = 2
ALPHA = 0.5

NC = 2
NS = 16
CH = 128
NB = 2
G = 8
NPAD = 10240
ROWS_PER_SUB = NPAD // NS
N_CHUNKS = -(-E // (NS * CH * NB)) * NB
EPC = N_CHUNKS * CH
E_PAD = EPC * NS
E_EXT = E_PAD + 8 * CH
NI = 2

_MESH = plsc.VectorSubcoreMesh(core_axis_name="c", subcore_axis_name="s",
                               num_cores=NC, num_subcores=NS)


def _mm_norm_body(x_ref, w_ref, b_ref, o_ref):
    y = jnp.dot(x_ref[...], w_ref[...], preferred_element_type=jnp.float32)
    y = y + b_ref[...]
    o_ref[...] = y * lax.rsqrt(jnp.sum(y * y, axis=1, keepdims=True) + 1e-12)


def _mm_norm(feats, w, b, bm):
    m, k = feats.shape
    return pl.pallas_call(
        _mm_norm_body,
        grid=(m // bm,),
        in_specs=[
            pl.BlockSpec((bm, k), lambda i: (i, 0)),
            pl.BlockSpec((k, D), lambda i: (0, 0)),
            pl.BlockSpec((1, D), lambda i: (0, 0)),
        ],
        out_specs=pl.BlockSpec((bm, D), lambda i: (i, 0)),
        out_shape=jax.ShapeDtypeStruct((m, D), jnp.float32),
    )(feats, w, b.reshape(1, D))


def _prescale_body(x_ref, deg_ref, o_ref):
    f = lax.rsqrt(jnp.maximum(deg_ref[...], 1.0))
    o_ref[...] = x_ref[...] * f


def _prescale(x, deg_col):
    bm = 2048
    return pl.pallas_call(
        _prescale_body,
        grid=(x.shape[0] // bm,),
        in_specs=[pl.BlockSpec((bm, D), lambda i: (i, 0)),
                  pl.BlockSpec((bm, 1), lambda i: (i, 0))],
        out_specs=pl.BlockSpec((bm, D), lambda i: (i, 0)),
        out_shape=jax.ShapeDtypeStruct(x.shape, jnp.float32),
    )(x, deg_col)


def _post_body(u_ref, x_ref, dd_ref, ds_ref, xn_ref, tn_ref):
    g = lax.rsqrt(jnp.maximum(dd_ref[...], 1.0))
    xn = u_ref[...] * g + ALPHA * x_ref[...]
    xn_ref[...] = xn
    tn_ref[...] = xn * lax.rsqrt(jnp.maximum(ds_ref[...], 1.0))


def _postscale(u, x, dd_col, ds_col):
    bm = 2048
    return pl.pallas_call(
        _post_body,
        grid=(x.shape[0] // bm,),
        in_specs=[pl.BlockSpec((bm, D), lambda i: (i, 0)),
                  pl.BlockSpec((bm, D), lambda i: (i, 0)),
                  pl.BlockSpec((bm, 1), lambda i: (i, 0)),
                  pl.BlockSpec((bm, 1), lambda i: (i, 0))],
        out_specs=[pl.BlockSpec((bm, D), lambda i: (i, 0)),
                   pl.BlockSpec((bm, D), lambda i: (i, 0))],
        out_shape=[jax.ShapeDtypeStruct(x.shape, jnp.float32),
                   jax.ShapeDtypeStruct(x.shape, jnp.float32)],
    )(u, x, dd_col, ds_col)


_DEG_SLOTS = 4


def _deg_kernel_body(idx2_hbm, ones_hbm, zeros_hbm, deg_hbm,
                     iv0, iv1, ones_v, acc, isem0, isem1):
    ivs = (iv0, iv1)
    isems = (isem0, isem1)
    c = lax.axis_index("c")
    s = lax.axis_index("s")
    ebase = pl.multiple_of(s * EPC, CH)
    pltpu.sync_copy(ones_hbm, ones_v)
    zb = pl.multiple_of(s * ROWS_PER_SUB, 128)
    pltpu.sync_copy(zeros_hbm, acc.at[pl.ds(zb, ROWS_PER_SUB)])
    for b in range(2):
        pltpu.async_copy(idx2_hbm.at[c, pl.ds(ebase + b * CH, CH)],
                         ivs[b], isems[b])
    plsc.subcore_barrier()

    @pl.loop(0, N_CHUNKS, step=2)
    def _(g):
        for b in range(2):
            pltpu.make_async_copy(idx2_hbm.at[c, pl.ds(0, CH)], ivs[b],
                                  isems[b]).wait()
            pltpu.sync_copy(ones_v, acc.at[ivs[b]], add=True)
            nxt = pl.multiple_of(ebase + (g + 2 + b) * CH, CH)
            pltpu.async_copy(idx2_hbm.at[c, pl.ds(nxt, CH)], ivs[b],
                             isems[b])

    for b in range(2):
        pltpu.make_async_copy(idx2_hbm.at[c, pl.ds(0, CH)], ivs[b],
                              isems[b]).wait()
    plsc.subcore_barrier()
    sl = pl.ds(zb, ROWS_PER_SUB)
    pltpu.sync_copy(acc.at[sl], deg_hbm.at[c, sl])


def _degrees(idx2, ones_vec, zeros_vec):
    k = pl.kernel(
        _deg_kernel_body,
        out_type=jax.ShapeDtypeStruct((NC, NPAD), jnp.float32),
        mesh=_MESH,
        scratch_types=[
            pltpu.VMEM((CH,), jnp.int32),
            pltpu.VMEM((CH,), jnp.int32),
            pltpu.VMEM((CH,), jnp.float32),
            pltpu.VMEM_SHARED((NPAD,), jnp.float32),
            pltpu.SemaphoreType.DMA,
            pltpu.SemaphoreType.DMA,
        ],
    )
    return k(idx2, ones_vec, zeros_vec)


def _spmm_kernel_body(t_hbm, src2_hbm, dst_hbm, zrows_hbm, u_hbm, *scr):
    svs = scr[0:NI]
    dvs = scr[NI:2 * NI]
    rows = scr[2 * NI]
    acc = scr[2 * NI + 1]
    isems = scr[2 * NI + 2:2 * NI + 2 + NI]
    dsems = scr[2 * NI + 2 + NI:2 * NI + 2 + 2 * NI]
    gsems = scr[2 * NI + 2 + 2 * NI:]
    c = lax.axis_index("c")
    s = lax.axis_index("s")
    ebase = pl.multiple_of(s * EPC, CH)
    abase = pl.multiple_of(s * ROWS_PER_SUB, 128)

    def refill(slot, q):
        nxt = pl.multiple_of(ebase + q * CH, CH)
        pltpu.async_copy(src2_hbm.at[c, pl.ds(nxt, CH)], svs[slot],
                         isems[slot])
        pltpu.async_copy(dst_hbm.at[pl.ds(nxt, CH)], dvs[slot],
                         dsems[slot])

    pltpu.sync_copy(zrows_hbm, rows.at[0])
    for r in range(ROWS_PER_SUB // CH):
        pltpu.sync_copy(rows.at[0], acc.at[pl.ds(abase + r * CH, CH)])
    for j in range(NI):
        refill(j, j)
    plsc.subcore_barrier()

    @pl.loop(0, N_CHUNKS, step=2)
    def _(g):
        for b in range(2):
            pltpu.make_async_copy(src2_hbm.at[c, pl.ds(0, CH)],
                                  svs[b], isems[b]).wait()
            pltpu.async_copy(t_hbm.at[svs[b]], rows.at[b], gsems[b])
        for b in range(2):
            pltpu.make_async_copy(t_hbm.at[svs[b]], rows.at[b],
                                  gsems[b]).wait()
            pltpu.make_async_copy(dst_hbm.at[pl.ds(0, CH)], dvs[b],
                                  dsems[b]).wait()
            pltpu.sync_copy(rows.at[b], acc.at[dvs[b]], add=True)
            refill(b, g + 2 + b)

    for j in range(NI):
        pltpu.make_async_copy(src2_hbm.at[c, pl.ds(0, CH)], svs[j],
                              isems[j]).wait()
        pltpu.make_async_copy(dst_hbm.at[pl.ds(0, CH)], dvs[j],
                              dsems[j]).wait()
    plsc.subcore_barrier()
    ob = pl.multiple_of(c * NPAD + s * ROWS_PER_SUB, 128)
    pltpu.sync_copy(acc.at[pl.ds(abase, ROWS_PER_SUB)],
                    u_hbm.at[pl.ds(ob, ROWS_PER_SUB)])


def _spmm(t_both, src2, dst_p, zrows):
    k = pl.kernel(
        _spmm_kernel_body,
        out_type=jax.ShapeDtypeStruct((NC * NPAD, D), jnp.float32),
        mesh=_MESH,
        scratch_types=(
            [pltpu.VMEM((CH,), jnp.int32) for _ in range(2 * NI)] + [
                pltpu.VMEM((2, CH, D), jnp.float32),
                pltpu.VMEM_SHARED((NPAD, D), jnp.float32),
            ] + [pltpu.SemaphoreType.DMA] * (2 * NI + 2)
        ),
    )
    return k(t_both, src2, dst_p, zrows)


def kernel(image_feats, text_feats, image_preference, text_preference,
           W_img, b_img, W_txt, b_txt, edge_index):
    src = edge_index[0]
    dst = edge_index[1]
    pad = E_EXT - E
    src_p = jnp.concatenate([src, jnp.full((pad,), N, jnp.int32)])
    dst_p = jnp.concatenate([dst, jnp.full((pad,), N, jnp.int32)])
    src2 = jnp.stack([src_p, src_p + NPAD])
    idx2 = jnp.stack([src_p, dst_p])

    ones_vec = jnp.ones((CH,), jnp.float32)
    zeros_vec = jnp.zeros((ROWS_PER_SUB,), jnp.float32)
    zrows = jnp.zeros((CH, D), jnp.float32)

    deg2 = _degrees(idx2, ones_vec, zeros_vec)
    deg_src = jnp.concatenate([deg2[0], deg2[0]]).reshape(NC * NPAD, 1)
    deg_dst = jnp.concatenate([deg2[1], deg2[1]]).reshape(NC * NPAD, 1)

    img_emb = _mm_norm(image_feats, W_img, b_img, bm=200)
    txt_emb = _mm_norm(text_feats, W_txt, b_txt, bm=200)

    zpad = jnp.zeros((NPAD - N, D), jnp.float32)
    x = jnp.concatenate([image_preference, img_emb, zpad,
                         text_preference, txt_emb, zpad], axis=0)

    t = _prescale(x, deg_src)
    for _layer in range(N_LAYERS):
        u = _spmm(t, src2, dst_p, zrows)
        x, t = _postscale(u, x, deg_dst, deg_src)

    user_preference = jnp.concatenate(
        [x[0:N_USERS], x[NPAD:NPAD + N_USERS]], axis=1)
    items = jnp.concatenate(
        [x[N_USERS:N], x[NPAD + N_USERS:NPAD + N]], axis=1)
    return (user_preference, items)


# speedup vs baseline: 8.5199x; 1.0005x over previous
import jax
import jax.numpy as jnp
from jax import lax
from jax.experimental import pallas as pl
from jax.experimental.pallas import tpu as pltpu
from jax.experimental.pallas import tpu_sc as plsc

N_USERS = 5000
N_ITEMS = 5000
N = N_USERS + N_ITEMS
D = 128
E = 320000
N_LAYERS = 2
ALPHA = 0.5

NC = 2
NS = 16
CH = 128
NB = 2
NPAD = 10240
ROWS_PER_SUB = NPAD // NS
N_CHUNKS = -(-E // (NS * CH * NB)) * NB
EPC = N_CHUNKS * CH
E_PAD = EPC * NS
E_EXT = E_PAD + 8 * CH
NI = 2

_MESH = plsc.VectorSubcoreMesh(core_axis_name="c", subcore_axis_name="s",
                               num_cores=NC, num_subcores=NS)


def _mm_norm_body(x_ref, w_ref, b_ref, o_ref):
    y = jnp.dot(x_ref[...], w_ref[...], preferred_element_type=jnp.float32)
    y = y + b_ref[...]
    o_ref[...] = y * lax.rsqrt(jnp.sum(y * y, axis=1, keepdims=True) + 1e-12)


def _mm_norm(feats, w, b, bm):
    m, k = feats.shape
    return pl.pallas_call(
        _mm_norm_body,
        grid=(m // bm,),
        in_specs=[
            pl.BlockSpec((bm, k), lambda i: (i, 0)),
            pl.BlockSpec((k, D), lambda i: (0, 0)),
            pl.BlockSpec((1, D), lambda i: (0, 0)),
        ],
        out_specs=pl.BlockSpec((bm, D), lambda i: (i, 0)),
        out_shape=jax.ShapeDtypeStruct((m, D), jnp.float32),
    )(feats, w, b.reshape(1, D))


def _prescale_body(x_ref, deg_ref, o_ref):
    f = lax.rsqrt(jnp.maximum(deg_ref[...], 1.0))
    o_ref[...] = x_ref[...] * f


def _prescale(x, deg_col):
    bm = 2048
    return pl.pallas_call(
        _prescale_body,
        grid=(x.shape[0] // bm,),
        in_specs=[pl.BlockSpec((bm, D), lambda i: (i, 0)),
                  pl.BlockSpec((bm, 1), lambda i: (i, 0))],
        out_specs=pl.BlockSpec((bm, D), lambda i: (i, 0)),
        out_shape=jax.ShapeDtypeStruct(x.shape, jnp.float32),
    )(x, deg_col)


def _post_body(u_ref, x_ref, dd_ref, ds_ref, xn_ref, tn_ref):
    g = lax.rsqrt(jnp.maximum(dd_ref[...], 1.0))
    xn = u_ref[...] * g + ALPHA * x_ref[...]
    xn_ref[...] = xn
    tn_ref[...] = xn * lax.rsqrt(jnp.maximum(ds_ref[...], 1.0))


def _postscale(u, x, dd_col, ds_col):
    bm = 2048
    return pl.pallas_call(
        _post_body,
        grid=(x.shape[0] // bm,),
        in_specs=[pl.BlockSpec((bm, D), lambda i: (i, 0)),
                  pl.BlockSpec((bm, D), lambda i: (i, 0)),
                  pl.BlockSpec((bm, 1), lambda i: (i, 0)),
                  pl.BlockSpec((bm, 1), lambda i: (i, 0))],
        out_specs=[pl.BlockSpec((bm, D), lambda i: (i, 0)),
                   pl.BlockSpec((bm, D), lambda i: (i, 0))],
        out_shape=[jax.ShapeDtypeStruct(x.shape, jnp.float32),
                   jax.ShapeDtypeStruct(x.shape, jnp.float32)],
    )(u, x, dd_col, ds_col)


def _deg_kernel_body(idx2_hbm, ones_hbm, zeros_hbm, deg_hbm,
                     iv0, iv1, ones_v, acc, isem0, isem1):
    ivs = (iv0, iv1)
    isems = (isem0, isem1)
    c = lax.axis_index("c")
    s = lax.axis_index("s")
    ebase = pl.multiple_of(s * EPC, CH)
    pltpu.sync_copy(ones_hbm, ones_v)
    zb = pl.multiple_of(s * ROWS_PER_SUB, 128)
    pltpu.sync_copy(zeros_hbm, acc.at[pl.ds(zb, ROWS_PER_SUB)])
    for b in range(2):
        pltpu.async_copy(idx2_hbm.at[c, pl.ds(ebase + b * CH, CH)],
                         ivs[b], isems[b])
    plsc.subcore_barrier()

    @pl.loop(0, N_CHUNKS, step=2)
    def _(g):
        for b in range(2):
            pltpu.make_async_copy(idx2_hbm.at[c, pl.ds(0, CH)], ivs[b],
                                  isems[b]).wait()
            pltpu.sync_copy(ones_v, acc.at[ivs[b]], add=True)
            nxt = pl.multiple_of(ebase + (g + 2 + b) * CH, CH)
            pltpu.async_copy(idx2_hbm.at[c, pl.ds(nxt, CH)], ivs[b],
                             isems[b])

    for b in range(2):
        pltpu.make_async_copy(idx2_hbm.at[c, pl.ds(0, CH)], ivs[b],
                              isems[b]).wait()
    plsc.subcore_barrier()
    sl = pl.ds(zb, ROWS_PER_SUB)
    pltpu.sync_copy(acc.at[sl], deg_hbm.at[c, sl])


def _degrees(idx2, ones_vec, zeros_vec):
    k = pl.kernel(
        _deg_kernel_body,
        out_type=jax.ShapeDtypeStruct((NC, NPAD), jnp.float32),
        mesh=_MESH,
        scratch_types=[
            pltpu.VMEM((CH,), jnp.int32),
            pltpu.VMEM((CH,), jnp.int32),
            pltpu.VMEM((CH,), jnp.float32),
            pltpu.VMEM_SHARED((NPAD,), jnp.float32),
            pltpu.SemaphoreType.DMA,
            pltpu.SemaphoreType.DMA,
        ],
    )
    return k(idx2, ones_vec, zeros_vec)


def _spmm_kernel_body(t_hbm, src2_hbm, dst_hbm, zrows_hbm, u_hbm, *scr):
    svs = scr[0:NI]
    dvs = scr[NI:2 * NI]
    rows = scr[2 * NI]
    acc = scr[2 * NI + 1]
    isems = scr[2 * NI + 2:2 * NI + 2 + NI]
    dsems = scr[2 * NI + 2 + NI:2 * NI + 2 + 2 * NI]
    gsems = scr[2 * NI + 2 + 2 * NI:]
    c = lax.axis_index("c")
    s = lax.axis_index("s")
    ebase = pl.multiple_of(s * EPC, CH)
    abase = pl.multiple_of(s * ROWS_PER_SUB, 128)

    def refill(slot, q):
        nxt = pl.multiple_of(ebase + q * CH, CH)
        pltpu.async_copy(src2_hbm.at[c, pl.ds(nxt, CH)], svs[slot],
                         isems[slot])
        pltpu.async_copy(dst_hbm.at[pl.ds(nxt, CH)], dvs[slot],
                         dsems[slot])

    pltpu.sync_copy(zrows_hbm, rows.at[0])
    for r in range(ROWS_PER_SUB // CH):
        pltpu.sync_copy(rows.at[0], acc.at[pl.ds(abase + r * CH, CH)])
    for j in range(NI):
        refill(j, j)
    plsc.subcore_barrier()

    @pl.loop(0, N_CHUNKS, step=2)
    def _(g):
        for b in range(2):
            pltpu.make_async_copy(src2_hbm.at[c, pl.ds(0, CH)],
                                  svs[b], isems[b]).wait()
            pltpu.async_copy(t_hbm.at[svs[b]], rows.at[b], gsems[b])
        for b in range(2):
            pltpu.make_async_copy(t_hbm.at[svs[b]], rows.at[b],
                                  gsems[b]).wait()
            pltpu.make_async_copy(dst_hbm.at[pl.ds(0, CH)], dvs[b],
                                  dsems[b]).wait()
            pltpu.sync_copy(rows.at[b], acc.at[dvs[b]], add=True)
            refill(b, g + 2 + b)

    for j in range(NI):
        pltpu.make_async_copy(src2_hbm.at[c, pl.ds(0, CH)], svs[j],
                              isems[j]).wait()
        pltpu.make_async_copy(dst_hbm.at[pl.ds(0, CH)], dvs[j],
                              dsems[j]).wait()
    plsc.subcore_barrier()
    ob = pl.multiple_of(c * NPAD + s * ROWS_PER_SUB, 128)
    pltpu.sync_copy(acc.at[pl.ds(abase, ROWS_PER_SUB)],
                    u_hbm.at[pl.ds(ob, ROWS_PER_SUB)])


def _spmm(t_both, src2, dst_p, zrows):
    k = pl.kernel(
        _spmm_kernel_body,
        out_type=jax.ShapeDtypeStruct((NC * NPAD, D), jnp.float32),
        mesh=_MESH,
        scratch_types=(
            [pltpu.VMEM((CH,), jnp.int32) for _ in range(2 * NI)] + [
                pltpu.VMEM((2, CH, D), jnp.float32),
                pltpu.VMEM_SHARED((NPAD, D), jnp.float32),
            ] + [pltpu.SemaphoreType.DMA] * (2 * NI + 2)
        ),
    )
    return k(t_both, src2, dst_p, zrows)


def kernel(image_feats, text_feats, image_preference, text_preference,
           W_img, b_img, W_txt, b_txt, edge_index):
    src = edge_index[0]
    dst = edge_index[1]
    pad = E_EXT - E
    src_p = jnp.concatenate([src, jnp.full((pad,), N, jnp.int32)])
    dst_p = jnp.concatenate([dst, jnp.full((pad,), N, jnp.int32)])
    src2 = jnp.stack([src_p, src_p + NPAD])
    idx2 = jnp.stack([src_p, dst_p])

    ones_vec = jnp.ones((CH,), jnp.float32)
    zeros_vec = jnp.zeros((ROWS_PER_SUB,), jnp.float32)
    zrows = jnp.zeros((CH, D), jnp.float32)

    deg2 = _degrees(idx2, ones_vec, zeros_vec)
    deg_src = jnp.concatenate([deg2[0], deg2[0]]).reshape(NC * NPAD, 1)
    deg_dst = jnp.concatenate([deg2[1], deg2[1]]).reshape(NC * NPAD, 1)

    img_emb = _mm_norm(image_feats, W_img, b_img, bm=200)
    txt_emb = _mm_norm(text_feats, W_txt, b_txt, bm=200)

    zpad = jnp.zeros((NPAD - N, D), jnp.float32)
    x = jnp.concatenate([image_preference, img_emb, zpad,
                         text_preference, txt_emb, zpad], axis=0)

    t = _prescale(x, deg_src)
    for _layer in range(N_LAYERS):
        u = _spmm(t, src2, dst_p, zrows)
        x, t = _postscale(u, x, deg_dst, deg_src)

    user_preference = jnp.concatenate(
        [x[0:N_USERS], x[NPAD:NPAD + N_USERS]], axis=1)
    items = jnp.concatenate(
        [x[N_USERS:N], x[NPAD + N_USERS:NPAD + N]], axis=1)
    return (user_preference, items)
